# Initial kernel scaffold; baseline (speedup 1.0000x reference)
import functools

import jax
import jax.numpy as jnp
from jax import lax
from jax.experimental import pallas as pl
from jax.experimental.pallas import tpu as pltpu
from jax.experimental.pallas import tpu_sc as plsc

N = 10000
E = 320000
H = 8
F = 128
HF = H * F
HP = 16
NPAD = 10240
NCORE = 2
NSUB = 16
C = 1280
CPC = (NPAD // C) // NCORE
ES = E // NSUB
ROWS_PT = C // NSUB
DEN_PT = NPAD // NSUB
BE = 32


def _feats_body(x_ref, w_ref, aal_ref, aar_ref, h_ref, el_ref, er_ref):
    hb = jnp.dot(x_ref[...], w_ref[...], preferred_element_type=jnp.float32)
    h_ref[...] = hb
    el_ref[...] = jnp.dot(hb, aal_ref[...], preferred_element_type=jnp.float32)
    er_ref[...] = jnp.dot(hb, aar_ref[...], preferred_element_type=jnp.float32)


def _feats(x, w, aal, aar, bn):
    n = x.shape[0]
    return pl.pallas_call(
        _feats_body,
        grid=(n // bn,),
        in_specs=[
            pl.BlockSpec((bn, F), lambda i: (i, 0)),
            pl.BlockSpec((F, HF), lambda i: (0, 0)),
            pl.BlockSpec((HF, HP), lambda i: (0, 0)),
            pl.BlockSpec((HF, HP), lambda i: (0, 0)),
        ],
        out_specs=[
            pl.BlockSpec((bn, HF), lambda i: (i, 0)),
            pl.BlockSpec((bn, HP), lambda i: (i, 0)),
            pl.BlockSpec((bn, HP), lambda i: (i, 0)),
        ],
        out_shape=[
            jax.ShapeDtypeStruct((n, HF), jnp.float32),
            jax.ShapeDtypeStruct((n, HP), jnp.float32),
            jax.ShapeDtypeStruct((n, HP), jnp.float32),
        ],
    )(x, w, aal, aar)


def _mid_body(acc_ref, d0_ref, d1_ref, bexp_ref, e8_ref, m_ref,
              w_ref, aal_ref, aar_ref, h_ref, el_ref, er_ref):
    dinv = 1.0 / (d0_ref[...] + d1_ref[...] + 1e-9)
    dexp = jnp.dot(dinv, e8_ref[...], preferred_element_type=jnp.float32)
    t = acc_ref[...] * dexp + bexp_ref[...]
    x1 = jnp.dot(t, m_ref[...], preferred_element_type=jnp.float32)
    x1 = jnp.where(x1 > 0.0, x1, jnp.exp(jnp.minimum(x1, 0.0)) - 1.0)
    hb = jnp.dot(x1, w_ref[...], preferred_element_type=jnp.float32)
    h_ref[...] = hb
    el_ref[...] = jnp.dot(hb, aal_ref[...], preferred_element_type=jnp.float32)
    er_ref[...] = jnp.dot(hb, aar_ref[...], preferred_element_type=jnp.float32)


def _mid(acc, d0, d1, bexp, e8, m, w, aal, aar, bn):
    n = acc.shape[0]
    return pl.pallas_call(
        _mid_body,
        grid=(n // bn,),
        in_specs=[
            pl.BlockSpec((bn, HF), lambda i: (i, 0)),
            pl.BlockSpec((bn, HP), lambda i: (i, 0)),
            pl.BlockSpec((bn, HP), lambda i: (i, 0)),
            pl.BlockSpec((1, HF), lambda i: (0, 0)),
            pl.BlockSpec((HP, HF), lambda i: (0, 0)),
            pl.BlockSpec((HF, F), lambda i: (0, 0)),
            pl.BlockSpec((F, HF), lambda i: (0, 0)),
            pl.BlockSpec((HF, HP), lambda i: (0, 0)),
            pl.BlockSpec((HF, HP), lambda i: (0, 0)),
        ],
        out_specs=[
            pl.BlockSpec((bn, HF), lambda i: (i, 0)),
            pl.BlockSpec((bn, HP), lambda i: (i, 0)),
            pl.BlockSpec((bn, HP), lambda i: (i, 0)),
        ],
        out_shape=[
            jax.ShapeDtypeStruct((n, HF), jnp.float32),
            jax.ShapeDtypeStruct((n, HP), jnp.float32),
            jax.ShapeDtypeStruct((n, HP), jnp.float32),
        ],
    )(acc, d0, d1, bexp, e8, m, w, aal, aar)


def _final_body(acc_ref, d0_ref, d1_ref, bexp_ref, e8_ref, m_ref, o_ref):
    dinv = 1.0 / (d0_ref[...] + d1_ref[...] + 1e-9)
    dexp = jnp.dot(dinv, e8_ref[...], preferred_element_type=jnp.float32)
    t = acc_ref[...] * dexp + bexp_ref[...]
    o_ref[...] = jnp.dot(t, m_ref[...], preferred_element_type=jnp.float32)


def _final(acc, d0, d1, bexp, e8, m, bn):
    n = acc.shape[0]
    return pl.pallas_call(
        _final_body,
        grid=(n // bn,),
        in_specs=[
            pl.BlockSpec((bn, HF), lambda i: (i, 0)),
            pl.BlockSpec((bn, HP), lambda i: (i, 0)),
            pl.BlockSpec((bn, HP), lambda i: (i, 0)),
            pl.BlockSpec((1, HF), lambda i: (0, 0)),
            pl.BlockSpec((HP, HF), lambda i: (0, 0)),
            pl.BlockSpec((HF, F), lambda i: (0, 0)),
        ],
        out_specs=pl.BlockSpec((bn, F), lambda i: (i, 0)),
        out_shape=jax.ShapeDtypeStruct((n, F), jnp.float32),
    )(acc, d0, d1, bexp, e8, m)


def _sc_body(edge_ref, el_ref, er_ref, h_ref, zden_ref,
             out_ref, den_ref,
             ssrc, sdst, match, hblk, elblk, erblk, wblk,
             srcb, dstb, dlocb, zrows, acc_sp, den_sp, sem_g, sem_h):
    c = lax.axis_index("c")
    s = lax.axis_index("s")
    iota16 = lax.iota(jnp.int32, 16)
    zero16 = jnp.zeros((16,), jnp.float32)

    base_e = s * ES
    pltpu.sync_copy(edge_ref.at[0, pl.ds(base_e, ES)], ssrc)
    pltpu.sync_copy(edge_ref.at[1, pl.ds(base_e, ES)], sdst)

    def _zrow(i, _):
        def _zcol(j, _):
            zrows[i, pl.ds(j * 16, 16)] = zero16
            return 0
        return lax.fori_loop(0, HF // 16, _zcol, 0)
    lax.fori_loop(0, 16, _zrow, 0)

    pltpu.sync_copy(zden_ref, den_sp.at[pl.ds(s * DEN_PT, DEN_PT)])

    def _chunk(k, _):
        lo = (c * CPC + k) * C

        for j in range(ROWS_PT // 16):
            pltpu.sync_copy(zrows, acc_sp.at[pl.ds(s * ROWS_PT + j * 16, 16)])
        plsc.subcore_barrier()

        def _filt(g, cnt):
            d16 = sdst[pl.ds(g * 16, 16)]
            m = (d16 >= lo) & (d16 < lo + C)
            mi = jnp.where(m, 1, 0).astype(jnp.int32)
            pos = cnt + plsc.cumsum(mi) - 1
            lid = g * 16 + iota16
            plsc.store_scatter(match, [pos], lid, mask=m)
            return cnt + jnp.sum(mi)
        count = lax.fori_loop(0, ES // 16, _filt, jnp.int32(0))

        nb = (count + (BE - 1)) // BE

        def _block(b, _):
            for half in range(BE // 16):
                off = b * BE + half * 16
                lid = match[pl.ds(off, 16)]
                valid = (off + iota16) < count
                lid = jnp.where(valid, lid, 0)
                s16 = plsc.load_gather(ssrc, [lid])
                d16 = plsc.load_gather(sdst, [lid])
                s16 = jnp.where(valid, s16, 0)
                d16 = jnp.where(valid, d16, lo)
                srcb[pl.ds(half * 16, 16)] = s16
                dstb[pl.ds(half * 16, 16)] = d16
                dlocb[pl.ds(half * 16, 16)] = d16 - lo
            cp_h = pltpu.async_copy(h_ref.at[srcb], hblk, sem_h)
            pltpu.async_copy(el_ref.at[srcb], elblk, sem_g).wait()
            pltpu.async_copy(er_ref.at[dstb], erblk, sem_g).wait()
            bbase = b * BE
            for e in range(BE):
                sv = elblk[e, :] + erblk[e, :]
                wv = jnp.exp(jnp.maximum(sv, 0.2 * sv))
                ok = (bbase + e) < count
                wblk[e, :] = jnp.where(ok, wv, zero16)
            pltpu.sync_copy(wblk, den_sp.at[dstb], add=True)
            cp_h.wait()

            def _scale(e, _):
                for hh in range(H):
                    ws = jnp.full((16,), wblk[e, hh], jnp.float32)
                    for j in range(F // 16):
                        o = hh * F + j * 16
                        hblk[e, pl.ds(o, 16)] = hblk[e, pl.ds(o, 16)] * ws
                return 0
            lax.fori_loop(0, BE, _scale, 0)

            pltpu.sync_copy(hblk, acc_sp.at[dlocb], add=True)
            return 0
        lax.fori_loop(0, nb, _block, 0)

        plsc.subcore_barrier()
        for j in range(ROWS_PT // 16):
            r = s * ROWS_PT + j * 16
            pltpu.sync_copy(acc_sp.at[pl.ds(r, 16)], out_ref.at[pl.ds(lo + r, 16)])
        return 0
    lax.fori_loop(0, CPC, _chunk, 0)

    plsc.subcore_barrier()
    pltpu.sync_copy(den_sp.at[pl.ds(s * DEN_PT, DEN_PT)],
                    den_ref.at[c, pl.ds(s * DEN_PT, DEN_PT)])


def _sc_edge(edge_index, el, er, h, zden):
    mesh = plsc.VectorSubcoreMesh(core_axis_name="c", subcore_axis_name="s")
    kern = pl.kernel(
        _sc_body,
        out_type=[
            jax.ShapeDtypeStruct((NPAD, HF), jnp.float32),
            jax.ShapeDtypeStruct((NCORE, NPAD, HP), jnp.float32),
        ],
        mesh=mesh,
        scratch_types=[
            pltpu.VMEM((ES,), jnp.int32),
            pltpu.VMEM((ES,), jnp.int32),
            pltpu.VMEM((ES,), jnp.int32),
            pltpu.VMEM((BE, HF), jnp.float32),
            pltpu.VMEM((BE, HP), jnp.float32),
            pltpu.VMEM((BE, HP), jnp.float32),
            pltpu.VMEM((BE, HP), jnp.float32),
            pltpu.VMEM((BE,), jnp.int32),
            pltpu.VMEM((BE,), jnp.int32),
            pltpu.VMEM((BE,), jnp.int32),
            pltpu.VMEM((16, HF), jnp.float32),
            pltpu.VMEM_SHARED((C, HF), jnp.float32),
            pltpu.VMEM_SHARED((NPAD, HP), jnp.float32),
            pltpu.SemaphoreType.DMA,
            pltpu.SemaphoreType.DMA,
        ],
    )
    return kern(edge_index, el, er, h, zden)


def _expand_att(a):
    eye = jnp.eye(HP, dtype=jnp.float32)[:H]
    return (a[:, :, None] * eye[:, None, :]).reshape(HF, HP)


def kernel(x, edge_index, W1, al1, ar1, b1, W2, al2, ar2, b2):
    aal1, aar1 = _expand_att(al1), _expand_att(ar1)
    aal2, aar2 = _expand_att(al2), _expand_att(ar2)
    e8 = jnp.zeros((HP, HF), jnp.float32).at[:H].set(
        jnp.repeat(jnp.eye(H, dtype=jnp.float32), F, axis=1))
    m = jnp.tile(jnp.eye(F, dtype=jnp.float32) / H, (H, 1))
    zden = jnp.zeros((DEN_PT, HP), jnp.float32)
    bexp1 = b1.reshape(1, HF)
    bexp2 = b2.reshape(1, HF)

    h1, el1, er1 = _feats(x, W1, aal1, aar1, bn=1000)
    acc1, den1 = _sc_edge(edge_index, el1, er1, h1, zden)
    h2, el2, er2 = _mid(acc1, den1[0], den1[1], bexp1, e8, m,
                        W2, aal2, aar2, bn=1024)
    acc2, den2 = _sc_edge(edge_index, el2, er2, h2, zden)
    out = _final(acc2, den2[0], den2[1], bexp2, e8, m, bn=1024)
    return out[:N]


# speedup vs baseline: 7.6376x; 7.6376x over previous
import functools

import jax
import jax.numpy as jnp
from jax import lax
from jax.experimental import pallas as pl
from jax.experimental.pallas import tpu as pltpu
from jax.experimental.pallas import tpu_sc as plsc

N = 10000
E = 320000
H = 8
F = 128
HF = H * F
HW = 128
NPAD = 10240
NCORE = 2
NSUB = 16
C = 512
CPC = (NPAD // C) // NCORE
ES = E // NSUB
ROWS_PT = C // NSUB
BE = 32
WG = 250
WE = WG * 16
NW = ES // WE
WCAP = 4096


def _feats_body(x_ref, w_ref, aal_ref, aar_ref, h_ref, el_ref, er_ref):
    hb = jnp.dot(x_ref[...], w_ref[...], preferred_element_type=jnp.float32)
    h_ref[...] = hb
    el_ref[...] = jnp.dot(hb, aal_ref[...], preferred_element_type=jnp.float32)
    er_ref[...] = jnp.dot(hb, aar_ref[...], preferred_element_type=jnp.float32)


def _feats(x, w, aal, aar, bn):
    n = x.shape[0]
    return pl.pallas_call(
        _feats_body,
        grid=(n // bn,),
        in_specs=[
            pl.BlockSpec((bn, F), lambda i: (i, 0)),
            pl.BlockSpec((F, HF), lambda i: (0, 0)),
            pl.BlockSpec((HF, HW), lambda i: (0, 0)),
            pl.BlockSpec((HF, HW), lambda i: (0, 0)),
        ],
        out_specs=[
            pl.BlockSpec((bn, HF), lambda i: (i, 0)),
            pl.BlockSpec((bn, HW), lambda i: (i, 0)),
            pl.BlockSpec((bn, HW), lambda i: (i, 0)),
        ],
        out_shape=[
            jax.ShapeDtypeStruct((n, HF), jnp.float32),
            jax.ShapeDtypeStruct((n, HW), jnp.float32),
            jax.ShapeDtypeStruct((n, HW), jnp.float32),
        ],
    )(x, w, aal, aar)


def _mid_body(acc_ref, den_ref, bexp_ref, e8_ref, m_ref,
              w_ref, aal_ref, aar_ref, h_ref, el_ref, er_ref):
    dinv = 1.0 / (den_ref[...] + 1e-9)
    dexp = jnp.dot(dinv, e8_ref[...], preferred_element_type=jnp.float32)
    t = acc_ref[...] * dexp + bexp_ref[...]
    x1 = jnp.dot(t, m_ref[...], preferred_element_type=jnp.float32)
    x1 = jnp.where(x1 > 0.0, x1, jnp.exp(jnp.minimum(x1, 0.0)) - 1.0)
    hb = jnp.dot(x1, w_ref[...], preferred_element_type=jnp.float32)
    h_ref[...] = hb
    el_ref[...] = jnp.dot(hb, aal_ref[...], preferred_element_type=jnp.float32)
    er_ref[...] = jnp.dot(hb, aar_ref[...], preferred_element_type=jnp.float32)


def _mid(acc, den, bexp, e8, m, w, aal, aar, bn):
    n = acc.shape[0]
    return pl.pallas_call(
        _mid_body,
        grid=(n // bn,),
        in_specs=[
            pl.BlockSpec((bn, HF), lambda i: (i, 0)),
            pl.BlockSpec((bn, HW), lambda i: (i, 0)),
            pl.BlockSpec((1, HF), lambda i: (0, 0)),
            pl.BlockSpec((HW, HF), lambda i: (0, 0)),
            pl.BlockSpec((HF, F), lambda i: (0, 0)),
            pl.BlockSpec((F, HF), lambda i: (0, 0)),
            pl.BlockSpec((HF, HW), lambda i: (0, 0)),
            pl.BlockSpec((HF, HW), lambda i: (0, 0)),
        ],
        out_specs=[
            pl.BlockSpec((bn, HF), lambda i: (i, 0)),
            pl.BlockSpec((bn, HW), lambda i: (i, 0)),
            pl.BlockSpec((bn, HW), lambda i: (i, 0)),
        ],
        out_shape=[
            jax.ShapeDtypeStruct((n, HF), jnp.float32),
            jax.ShapeDtypeStruct((n, HW), jnp.float32),
            jax.ShapeDtypeStruct((n, HW), jnp.float32),
        ],
    )(acc, den, bexp, e8, m, w, aal, aar)


def _final_body(acc_ref, den_ref, bexp_ref, e8_ref, m_ref, o_ref):
    dinv = 1.0 / (den_ref[...] + 1e-9)
    dexp = jnp.dot(dinv, e8_ref[...], preferred_element_type=jnp.float32)
    t = acc_ref[...] * dexp + bexp_ref[...]
    o_ref[...] = jnp.dot(t, m_ref[...], preferred_element_type=jnp.float32)


def _final(acc, den, bexp, e8, m, bn):
    n = acc.shape[0]
    return pl.pallas_call(
        _final_body,
        grid=(n // bn,),
        in_specs=[
            pl.BlockSpec((bn, HF), lambda i: (i, 0)),
            pl.BlockSpec((bn, HW), lambda i: (i, 0)),
            pl.BlockSpec((1, HF), lambda i: (0, 0)),
            pl.BlockSpec((HW, HF), lambda i: (0, 0)),
            pl.BlockSpec((HF, F), lambda i: (0, 0)),
        ],
        out_specs=pl.BlockSpec((bn, F), lambda i: (i, 0)),
        out_shape=jax.ShapeDtypeStruct((n, F), jnp.float32),
    )(acc, den, bexp, e8, m)


def _sc_body(edge_ref, el_ref, er_ref, hflat_ref,
             out_ref, den_ref,
             wsrc, wdst, match, sbuf, elblk, erblk, wblk,
             srcb, dstb, dlocb, gidxb, didxb, zden, acc_sp, den_sp,
             sem_g, sem_h):
    c = lax.axis_index("c")
    s = lax.axis_index("s")
    iota16 = lax.iota(jnp.int32, 16)
    zero16 = jnp.zeros((16,), jnp.float32)
    lane_is_head = iota16 < H
    base_e = s * ES

    def _zd(i, _):
        def _zc(j, _):
            zden[i, pl.ds(j * 16, 16)] = zero16
            return 0
        return lax.fori_loop(0, HW // 16, _zc, 0)
    lax.fori_loop(0, ROWS_PT, _zd, 0)

    def _zw(i, _):
        def _zc(j, _):
            wblk[i, pl.ds(j * 16, 16)] = zero16
            return 0
        return lax.fori_loop(0, HW // 16, _zc, 0)
    lax.fori_loop(0, BE, _zw, 0)

    def _chunk(k, _):
        lo = (c * CPC + k) * C

        for j in range(H):
            pltpu.sync_copy(zden, acc_sp.at[pl.ds(j * C + s * ROWS_PT, ROWS_PT)])
        pltpu.sync_copy(zden, den_sp.at[pl.ds(s * ROWS_PT, ROWS_PT)])
        plsc.subcore_barrier()

        def _wave(wv_i, _):
            woff = base_e + wv_i * WE
            pltpu.sync_copy(edge_ref.at[pl.ds(woff, WE)], wsrc)
            pltpu.sync_copy(edge_ref.at[pl.ds(E + woff, WE)], wdst)

            def _filt(g, cnt):
                d16 = wdst[pl.ds(g * 16, 16)]
                m = (d16 >= lo) & (d16 < lo + C)
                mi = jnp.where(m, 1, 0).astype(jnp.int32)
                cum = plsc.cumsum(mi)
                pos = cnt + cum - 1
                lid = g * 16 + iota16
                plsc.store_scatter(match, [pos], lid, mask=m)
                return cnt + cum[15]
            count = lax.fori_loop(0, WG, _filt, jnp.int32(0))

            nb = (count + (BE - 1)) // BE

            def _block(b, _):
                for half in range(BE // 16):
                    off = b * BE + half * 16
                    lid = match[pl.ds(off, 16)]
                    valid = (off + iota16) < count
                    lid = jnp.where(valid, lid, 0)
                    s16 = plsc.load_gather(wsrc, [lid])
                    d16 = plsc.load_gather(wdst, [lid])
                    s16 = jnp.where(valid, s16, 0)
                    d16 = jnp.where(valid, d16, lo)
                    srcb[pl.ds(half * 16, 16)] = s16
                    dstb[pl.ds(half * 16, 16)] = d16
                    dlocb[pl.ds(half * 16, 16)] = d16 - lo
                pltpu.async_copy(el_ref.at[srcb], elblk, sem_g).wait()
                pltpu.async_copy(er_ref.at[dstb], erblk, sem_g).wait()
                bbase = b * BE
                for e in range(BE):
                    sv = elblk[e, pl.ds(0, 16)] + erblk[e, pl.ds(0, 16)]
                    wv = jnp.exp(jnp.maximum(sv, 0.2 * sv))
                    ok = jnp.logical_and((bbase + e) < count, lane_is_head)
                    wblk[e, pl.ds(0, 16)] = jnp.where(ok, wv, zero16)
                pltpu.sync_copy(wblk, den_sp.at[dlocb], add=True)

                for j in range(H):
                    gidxb[pl.ds(0, 16)] = srcb[pl.ds(0, 16)] + j * N
                    gidxb[pl.ds(16, 16)] = srcb[pl.ds(16, 16)] + j * N
                    pltpu.async_copy(hflat_ref.at[gidxb], sbuf, sem_h).wait()
                    didxb[pl.ds(0, 16)] = dlocb[pl.ds(0, 16)] + j * C
                    didxb[pl.ds(16, 16)] = dlocb[pl.ds(16, 16)] + j * C

                    def _sc(e, _):
                        ws = jnp.full((16,), wblk[e, pl.ds(0, 16)][j], jnp.float32)
                        for q in range(F // 16):
                            o = q * 16
                            sbuf[e, pl.ds(o, 16)] = sbuf[e, pl.ds(o, 16)] * ws
                        return 0
                    lax.fori_loop(0, BE, _sc, 0)
                    pltpu.sync_copy(sbuf, acc_sp.at[didxb], add=True)
                return 0
            lax.fori_loop(0, nb, _block, 0)
            return 0
        lax.fori_loop(0, NW, _wave, 0)

        plsc.subcore_barrier()
        for j in range(H):
            pltpu.sync_copy(acc_sp.at[pl.ds(j * C + s * ROWS_PT, ROWS_PT)],
                            out_ref.at[j, pl.ds(lo + s * ROWS_PT, ROWS_PT)])
        pltpu.sync_copy(den_sp.at[pl.ds(s * ROWS_PT, ROWS_PT)],
                        den_ref.at[pl.ds(lo + s * ROWS_PT, ROWS_PT)])
        return 0
    lax.fori_loop(0, CPC, _chunk, 0)


def _sc_edge(edge_index, el, er, hflat):
    mesh = plsc.VectorSubcoreMesh(core_axis_name="c", subcore_axis_name="s",
                                  num_cores=NCORE, num_subcores=NSUB)
    kern = pl.kernel(
        _sc_body,
        out_type=[
            jax.ShapeDtypeStruct((H, NPAD, HW), jnp.float32),
            jax.ShapeDtypeStruct((NPAD, HW), jnp.float32),
        ],
        mesh=mesh,
        scratch_types=[
            pltpu.VMEM((WE,), jnp.int32),
            pltpu.VMEM((WE,), jnp.int32),
            pltpu.VMEM((WCAP,), jnp.int32),
            pltpu.VMEM((BE, HW), jnp.float32),
            pltpu.VMEM((BE, HW), jnp.float32),
            pltpu.VMEM((BE, HW), jnp.float32),
            pltpu.VMEM((BE, HW), jnp.float32),
            pltpu.VMEM((BE,), jnp.int32),
            pltpu.VMEM((BE,), jnp.int32),
            pltpu.VMEM((BE,), jnp.int32),
            pltpu.VMEM((BE,), jnp.int32),
            pltpu.VMEM((BE,), jnp.int32),
            pltpu.VMEM((ROWS_PT, HW), jnp.float32),
            pltpu.VMEM_SHARED((H * C, HW), jnp.float32),
            pltpu.VMEM_SHARED((C, HW), jnp.float32),
            pltpu.SemaphoreType.DMA,
            pltpu.SemaphoreType.DMA,
        ],
        compiler_params=pltpu.CompilerParams(needs_layout_passes=False),
    )
    return kern(edge_index.reshape(2 * E), el, er, hflat)


def _expand_att(a):
    eye = jnp.eye(HW, dtype=jnp.float32)[:H]
    return (a[:, :, None] * eye[:, None, :]).reshape(HF, HW)


def kernel(x, edge_index, W1, al1, ar1, b1, W2, al2, ar2, b2):
    aal1, aar1 = _expand_att(al1), _expand_att(ar1)
    aal2, aar2 = _expand_att(al2), _expand_att(ar2)
    e8 = jnp.zeros((HW, HF), jnp.float32).at[:H].set(
        jnp.repeat(jnp.eye(H, dtype=jnp.float32), F, axis=1))
    m = jnp.tile(jnp.eye(F, dtype=jnp.float32) / H, (H, 1))
    bexp1 = b1.reshape(1, HF)
    bexp2 = b2.reshape(1, HF)

    h1, el1, er1 = _feats(x, W1, aal1, aar1, bn=1000)
    h1f = jnp.transpose(h1.reshape(N, H, F), (1, 0, 2)).reshape(H * N, F)
    acc1, den1 = _sc_edge(edge_index, el1, er1, h1f)
    acc1t = jnp.transpose(acc1, (1, 0, 2)).reshape(NPAD, HF)
    h2, el2, er2 = _mid(acc1t, den1, bexp1, e8, m, W2, aal2, aar2, bn=1024)
    h2f = jnp.transpose(h2.reshape(NPAD, H, F)[:N], (1, 0, 2)).reshape(H * N, F)
    acc2, den2 = _sc_edge(edge_index, el2, er2, h2f)
    acc2t = jnp.transpose(acc2, (1, 0, 2)).reshape(NPAD, HF)
    out = _final(acc2t, den2, bexp2, e8, m, bn=1024)
    return out[:N]


# speedup vs baseline: 15.8986x; 2.0816x over previous
import functools

import jax
import jax.numpy as jnp
from jax import lax
from jax.experimental import pallas as pl
from jax.experimental.pallas import tpu as pltpu
from jax.experimental.pallas import tpu_sc as plsc

N = 10000
E = 320000
H = 8
F = 128
HF = H * F
HW = 128
NPAD = 10240
NCORE = 2
NSUB = 16
C = 512
CPC = (NPAD // C) // NCORE
ES = E // NSUB
ROWS_PT = C // NSUB
BE = 32
WG = 250
WE = WG * 16
NW = ES // WE
WCAP = 4096


def _feats_body(x_ref, w_ref, aal_ref, aar_ref, h_ref, el_ref, er_ref):
    hb = jnp.dot(x_ref[...], w_ref[...], preferred_element_type=jnp.float32)
    h_ref[...] = hb
    el_ref[...] = jnp.dot(hb, aal_ref[...], preferred_element_type=jnp.float32)
    er_ref[...] = jnp.dot(hb, aar_ref[...], preferred_element_type=jnp.float32)


def _feats(x, w, aal, aar, bn):
    n = x.shape[0]
    return pl.pallas_call(
        _feats_body,
        grid=(n // bn,),
        in_specs=[
            pl.BlockSpec((bn, F), lambda i: (i, 0)),
            pl.BlockSpec((F, HF), lambda i: (0, 0)),
            pl.BlockSpec((HF, HW), lambda i: (0, 0)),
            pl.BlockSpec((HF, HW), lambda i: (0, 0)),
        ],
        out_specs=[
            pl.BlockSpec((bn, HF), lambda i: (i, 0)),
            pl.BlockSpec((bn, HW), lambda i: (i, 0)),
            pl.BlockSpec((bn, HW), lambda i: (i, 0)),
        ],
        out_shape=[
            jax.ShapeDtypeStruct((n, HF), jnp.float32),
            jax.ShapeDtypeStruct((n, HW), jnp.float32),
            jax.ShapeDtypeStruct((n, HW), jnp.float32),
        ],
    )(x, w, aal, aar)


def _mid_body(acc_ref, den_ref, bexp_ref, e8_ref, m_ref,
              w_ref, aal_ref, aar_ref, h_ref, el_ref, er_ref):
    dinv = 1.0 / (den_ref[...] + 1e-9)
    dexp = jnp.dot(dinv, e8_ref[...], preferred_element_type=jnp.float32)
    t = acc_ref[...] * dexp + bexp_ref[...]
    x1 = jnp.dot(t, m_ref[...], preferred_element_type=jnp.float32)
    x1 = jnp.where(x1 > 0.0, x1, jnp.exp(jnp.minimum(x1, 0.0)) - 1.0)
    hb = jnp.dot(x1, w_ref[...], preferred_element_type=jnp.float32)
    h_ref[...] = hb
    el_ref[...] = jnp.dot(hb, aal_ref[...], preferred_element_type=jnp.float32)
    er_ref[...] = jnp.dot(hb, aar_ref[...], preferred_element_type=jnp.float32)


def _mid(acc, den, bexp, e8, m, w, aal, aar, bn):
    n = acc.shape[0]
    return pl.pallas_call(
        _mid_body,
        grid=(n // bn,),
        in_specs=[
            pl.BlockSpec((bn, HF), lambda i: (i, 0)),
            pl.BlockSpec((bn, HW), lambda i: (i, 0)),
            pl.BlockSpec((1, HF), lambda i: (0, 0)),
            pl.BlockSpec((HW, HF), lambda i: (0, 0)),
            pl.BlockSpec((HF, F), lambda i: (0, 0)),
            pl.BlockSpec((F, HF), lambda i: (0, 0)),
            pl.BlockSpec((HF, HW), lambda i: (0, 0)),
            pl.BlockSpec((HF, HW), lambda i: (0, 0)),
        ],
        out_specs=[
            pl.BlockSpec((bn, HF), lambda i: (i, 0)),
            pl.BlockSpec((bn, HW), lambda i: (i, 0)),
            pl.BlockSpec((bn, HW), lambda i: (i, 0)),
        ],
        out_shape=[
            jax.ShapeDtypeStruct((n, HF), jnp.float32),
            jax.ShapeDtypeStruct((n, HW), jnp.float32),
            jax.ShapeDtypeStruct((n, HW), jnp.float32),
        ],
    )(acc, den, bexp, e8, m, w, aal, aar)


def _final_body(acc_ref, den_ref, bexp_ref, e8_ref, m_ref, o_ref):
    dinv = 1.0 / (den_ref[...] + 1e-9)
    dexp = jnp.dot(dinv, e8_ref[...], preferred_element_type=jnp.float32)
    t = acc_ref[...] * dexp + bexp_ref[...]
    o_ref[...] = jnp.dot(t, m_ref[...], preferred_element_type=jnp.float32)


def _final(acc, den, bexp, e8, m, bn):
    n = acc.shape[0]
    return pl.pallas_call(
        _final_body,
        grid=(n // bn,),
        in_specs=[
            pl.BlockSpec((bn, HF), lambda i: (i, 0)),
            pl.BlockSpec((bn, HW), lambda i: (i, 0)),
            pl.BlockSpec((1, HF), lambda i: (0, 0)),
            pl.BlockSpec((HW, HF), lambda i: (0, 0)),
            pl.BlockSpec((HF, F), lambda i: (0, 0)),
        ],
        out_specs=pl.BlockSpec((bn, F), lambda i: (i, 0)),
        out_shape=jax.ShapeDtypeStruct((n, F), jnp.float32),
    )(acc, den, bexp, e8, m)


def _sc_body(edge_ref, el_ref, er_ref, hflat_ref,
             out_ref, den_ref,
             wsrc, wdst, match, sbuf0, sbuf1, elblk, erblk, wblk,
             srcb, dstb, dlocb, gidxb0, gidxb1, didxb0, didxb1,
             zden, acc_sp, den_sp, sem_g, sem_h):
    sbuf = (sbuf0, sbuf1)
    gidxb = (gidxb0, gidxb1)
    didxb = (didxb0, didxb1)
    c = lax.axis_index("c")
    s = lax.axis_index("s")
    iota16 = lax.iota(jnp.int32, 16)
    zero16 = jnp.zeros((16,), jnp.float32)
    lane_is_head = iota16 < H
    base_e = s * ES

    def _zd(i, _):
        def _zc(j, _):
            zden[i, pl.ds(j * 16, 16)] = zero16
            return 0
        return lax.fori_loop(0, HW // 16, _zc, 0)
    lax.fori_loop(0, ROWS_PT, _zd, 0)

    def _zw(i, _):
        def _zc(j, _):
            wblk[i, pl.ds(j * 16, 16)] = zero16
            return 0
        return lax.fori_loop(0, HW // 16, _zc, 0)
    lax.fori_loop(0, BE, _zw, 0)

    def _chunk(k, _):
        lo = (c * CPC + k) * C

        for j in range(H):
            pltpu.sync_copy(zden, acc_sp.at[pl.ds(j * C + s * ROWS_PT, ROWS_PT)])
        pltpu.sync_copy(zden, den_sp.at[pl.ds(s * ROWS_PT, ROWS_PT)])
        plsc.subcore_barrier()

        def _wave(wv_i, _):
            woff = base_e + wv_i * WE
            pltpu.sync_copy(edge_ref.at[pl.ds(woff, WE)], wsrc)
            pltpu.sync_copy(edge_ref.at[pl.ds(E + woff, WE)], wdst)

            def _filt(g, cnt):
                d16 = wdst[pl.ds(g * 16, 16)]
                m = (d16 >= lo) & (d16 < lo + C)
                mi = jnp.where(m, 1, 0).astype(jnp.int32)
                cum = plsc.cumsum(mi)
                pos = cnt + cum - 1
                lid = g * 16 + iota16
                plsc.store_scatter(match, [pos], lid, mask=m)
                return cnt + cum[15]
            count = lax.fori_loop(0, WG, _filt, jnp.int32(0))

            nb = (count + (BE - 1)) // BE

            def _block(b, _):
                for half in range(BE // 16):
                    off = b * BE + half * 16
                    lid = match[pl.ds(off, 16)]
                    valid = (off + iota16) < count
                    lid = jnp.where(valid, lid, 0)
                    s16 = plsc.load_gather(wsrc, [lid])
                    d16 = plsc.load_gather(wdst, [lid])
                    s16 = jnp.where(valid, s16, 0)
                    d16 = jnp.where(valid, d16, lo)
                    srcb[pl.ds(half * 16, 16)] = s16
                    dstb[pl.ds(half * 16, 16)] = d16
                    dlocb[pl.ds(half * 16, 16)] = d16 - lo
                for g in range(2):
                    for j in range(4):
                        hj = g * 4 + j
                        gi = gidxb[g]
                        di = didxb[g]
                        gi[pl.ds(j * BE, 16)] = srcb[pl.ds(0, 16)] + hj * N
                        gi[pl.ds(j * BE + 16, 16)] = srcb[pl.ds(16, 16)] + hj * N
                        di[pl.ds(j * BE, 16)] = dlocb[pl.ds(0, 16)] + hj * C
                        di[pl.ds(j * BE + 16, 16)] = dlocb[pl.ds(16, 16)] + hj * C
                cp0 = pltpu.async_copy(hflat_ref.at[gidxb[0]], sbuf[0], sem_h)
                cp1 = pltpu.async_copy(hflat_ref.at[gidxb[1]], sbuf[1], sem_h)
                cpe = pltpu.async_copy(el_ref.at[srcb], elblk, sem_g)
                cpr = pltpu.async_copy(er_ref.at[dstb], erblk, sem_g)
                cpe.wait()
                cpr.wait()
                bbase = b * BE
                for e in range(BE):
                    sv = elblk[e, pl.ds(0, 16)] + erblk[e, pl.ds(0, 16)]
                    wv = jnp.exp(jnp.maximum(sv, 0.2 * sv))
                    ok = jnp.logical_and((bbase + e) < count, lane_is_head)
                    wblk[e, pl.ds(0, 16)] = jnp.where(ok, wv, zero16)
                pltpu.sync_copy(wblk, den_sp.at[dlocb], add=True)

                for g in range(2):
                    (cp0 if g == 0 else cp1).wait()
                    sb = sbuf[g]
                    for j in range(4):
                        hj = g * 4 + j

                        def _sc(e, _):
                            ws = jnp.full((16,), wblk[e, pl.ds(0, 16)][hj],
                                          jnp.float32)
                            for q in range(F // 16):
                                o = q * 16
                                sb[j * BE + e, pl.ds(o, 16)] = (
                                    sb[j * BE + e, pl.ds(o, 16)] * ws)
                            return 0
                        lax.fori_loop(0, BE, _sc, 0)
                    pltpu.sync_copy(sb, acc_sp.at[didxb[g]], add=True)
                return 0
            lax.fori_loop(0, nb, _block, 0)
            return 0
        lax.fori_loop(0, NW, _wave, 0)

        plsc.subcore_barrier()
        for j in range(H):
            pltpu.sync_copy(acc_sp.at[pl.ds(j * C + s * ROWS_PT, ROWS_PT)],
                            out_ref.at[j, pl.ds(lo + s * ROWS_PT, ROWS_PT)])
        pltpu.sync_copy(den_sp.at[pl.ds(s * ROWS_PT, ROWS_PT)],
                        den_ref.at[pl.ds(lo + s * ROWS_PT, ROWS_PT)])
        return 0
    lax.fori_loop(0, CPC, _chunk, 0)


def _sc_edge(edge_index, el, er, hflat):
    mesh = plsc.VectorSubcoreMesh(core_axis_name="c", subcore_axis_name="s",
                                  num_cores=NCORE, num_subcores=NSUB)
    kern = pl.kernel(
        _sc_body,
        out_type=[
            jax.ShapeDtypeStruct((H, NPAD, HW), jnp.float32),
            jax.ShapeDtypeStruct((NPAD, HW), jnp.float32),
        ],
        mesh=mesh,
        scratch_types=[
            pltpu.VMEM((WE,), jnp.int32),
            pltpu.VMEM((WE,), jnp.int32),
            pltpu.VMEM((WCAP,), jnp.int32),
            pltpu.VMEM((4 * BE, HW), jnp.float32),
            pltpu.VMEM((4 * BE, HW), jnp.float32),
            pltpu.VMEM((BE, HW), jnp.float32),
            pltpu.VMEM((BE, HW), jnp.float32),
            pltpu.VMEM((BE, HW), jnp.float32),
            pltpu.VMEM((BE,), jnp.int32),
            pltpu.VMEM((BE,), jnp.int32),
            pltpu.VMEM((BE,), jnp.int32),
            pltpu.VMEM((4 * BE,), jnp.int32),
            pltpu.VMEM((4 * BE,), jnp.int32),
            pltpu.VMEM((4 * BE,), jnp.int32),
            pltpu.VMEM((4 * BE,), jnp.int32),
            pltpu.VMEM((ROWS_PT, HW), jnp.float32),
            pltpu.VMEM_SHARED((H * C, HW), jnp.float32),
            pltpu.VMEM_SHARED((C, HW), jnp.float32),
            pltpu.SemaphoreType.DMA,
            pltpu.SemaphoreType.DMA,
        ],
        compiler_params=pltpu.CompilerParams(needs_layout_passes=False),
    )
    return kern(edge_index.reshape(2 * E), el, er, hflat)


def _expand_att(a):
    eye = jnp.eye(HW, dtype=jnp.float32)[:H]
    return (a[:, :, None] * eye[:, None, :]).reshape(HF, HW)


def kernel(x, edge_index, W1, al1, ar1, b1, W2, al2, ar2, b2):
    aal1, aar1 = _expand_att(al1), _expand_att(ar1)
    aal2, aar2 = _expand_att(al2), _expand_att(ar2)
    e8 = jnp.zeros((HW, HF), jnp.float32).at[:H].set(
        jnp.repeat(jnp.eye(H, dtype=jnp.float32), F, axis=1))
    m = jnp.tile(jnp.eye(F, dtype=jnp.float32) / H, (H, 1))
    bexp1 = b1.reshape(1, HF)
    bexp2 = b2.reshape(1, HF)

    h1, el1, er1 = _feats(x, W1, aal1, aar1, bn=1000)
    h1f = jnp.transpose(h1.reshape(N, H, F), (1, 0, 2)).reshape(H * N, F)
    acc1, den1 = _sc_edge(edge_index, el1, er1, h1f)
    acc1t = jnp.transpose(acc1, (1, 0, 2)).reshape(NPAD, HF)
    h2, el2, er2 = _mid(acc1t, den1, bexp1, e8, m, W2, aal2, aar2, bn=1024)
    h2f = jnp.transpose(h2.reshape(NPAD, H, F)[:N], (1, 0, 2)).reshape(H * N, F)
    acc2, den2 = _sc_edge(edge_index, el2, er2, h2f)
    acc2t = jnp.transpose(acc2, (1, 0, 2)).reshape(NPAD, HF)
    out = _final(acc2t, den2, bexp2, e8, m, bn=1024)
    return out[:N]


# speedup vs baseline: 16.7863x; 1.0558x over previous
import functools

import jax
import jax.numpy as jnp
from jax import lax
from jax.experimental import pallas as pl
from jax.experimental.pallas import tpu as pltpu
from jax.experimental.pallas import tpu_sc as plsc

N = 10000
E = 320000
H = 8
F = 128
HF = H * F
HW = 128
NPAD = 10240
NCORE = 2
NSUB = 16
C = 512
CPC = (NPAD // C) // NCORE
ES = E // NSUB
ROWS_PT = C // NSUB
BE = 32
WG = 250
WE = WG * 16
NW = ES // WE
WCAP = 4096


def _feats_body(x_ref, w_ref, aal_ref, aar_ref, h_ref, el_ref, er_ref):
    hb = jnp.dot(x_ref[...], w_ref[...], preferred_element_type=jnp.float32)
    h_ref[...] = hb
    el_ref[...] = jnp.dot(hb, aal_ref[...], preferred_element_type=jnp.float32)
    er_ref[...] = jnp.dot(hb, aar_ref[...], preferred_element_type=jnp.float32)


def _feats(x, w, aal, aar, bn):
    n = x.shape[0]
    return pl.pallas_call(
        _feats_body,
        grid=(n // bn,),
        in_specs=[
            pl.BlockSpec((bn, F), lambda i: (i, 0)),
            pl.BlockSpec((F, HF), lambda i: (0, 0)),
            pl.BlockSpec((HF, HW), lambda i: (0, 0)),
            pl.BlockSpec((HF, HW), lambda i: (0, 0)),
        ],
        out_specs=[
            pl.BlockSpec((bn, HF), lambda i: (i, 0)),
            pl.BlockSpec((bn, HW), lambda i: (i, 0)),
            pl.BlockSpec((bn, HW), lambda i: (i, 0)),
        ],
        out_shape=[
            jax.ShapeDtypeStruct((n, HF), jnp.float32),
            jax.ShapeDtypeStruct((n, HW), jnp.float32),
            jax.ShapeDtypeStruct((n, HW), jnp.float32),
        ],
    )(x, w, aal, aar)


def _mid_body(acc_ref, den_ref, bexp_ref, e8_ref, m_ref,
              w_ref, aal_ref, aar_ref, h_ref, el_ref, er_ref):
    dinv = 1.0 / (den_ref[...] + 1e-9)
    dexp = jnp.dot(dinv, e8_ref[...], preferred_element_type=jnp.float32)
    t = acc_ref[...] * dexp + bexp_ref[...]
    x1 = jnp.dot(t, m_ref[...], preferred_element_type=jnp.float32)
    x1 = jnp.where(x1 > 0.0, x1, jnp.exp(jnp.minimum(x1, 0.0)) - 1.0)
    hb = jnp.dot(x1, w_ref[...], preferred_element_type=jnp.float32)
    h_ref[...] = hb
    el_ref[...] = jnp.dot(hb, aal_ref[...], preferred_element_type=jnp.float32)
    er_ref[...] = jnp.dot(hb, aar_ref[...], preferred_element_type=jnp.float32)


def _mid(acc, den, bexp, e8, m, w, aal, aar, bn):
    n = acc.shape[0]
    return pl.pallas_call(
        _mid_body,
        grid=(n // bn,),
        in_specs=[
            pl.BlockSpec((bn, HF), lambda i: (i, 0)),
            pl.BlockSpec((bn, HW), lambda i: (i, 0)),
            pl.BlockSpec((1, HF), lambda i: (0, 0)),
            pl.BlockSpec((HW, HF), lambda i: (0, 0)),
            pl.BlockSpec((HF, F), lambda i: (0, 0)),
            pl.BlockSpec((F, HF), lambda i: (0, 0)),
            pl.BlockSpec((HF, HW), lambda i: (0, 0)),
            pl.BlockSpec((HF, HW), lambda i: (0, 0)),
        ],
        out_specs=[
            pl.BlockSpec((bn, HF), lambda i: (i, 0)),
            pl.BlockSpec((bn, HW), lambda i: (i, 0)),
            pl.BlockSpec((bn, HW), lambda i: (i, 0)),
        ],
        out_shape=[
            jax.ShapeDtypeStruct((n, HF), jnp.float32),
            jax.ShapeDtypeStruct((n, HW), jnp.float32),
            jax.ShapeDtypeStruct((n, HW), jnp.float32),
        ],
    )(acc, den, bexp, e8, m, w, aal, aar)


def _final_body(acc_ref, den_ref, bexp_ref, e8_ref, m_ref, o_ref):
    dinv = 1.0 / (den_ref[...] + 1e-9)
    dexp = jnp.dot(dinv, e8_ref[...], preferred_element_type=jnp.float32)
    t = acc_ref[...] * dexp + bexp_ref[...]
    o_ref[...] = jnp.dot(t, m_ref[...], preferred_element_type=jnp.float32)


def _final(acc, den, bexp, e8, m, bn):
    n = acc.shape[0]
    return pl.pallas_call(
        _final_body,
        grid=(n // bn,),
        in_specs=[
            pl.BlockSpec((bn, HF), lambda i: (i, 0)),
            pl.BlockSpec((bn, HW), lambda i: (i, 0)),
            pl.BlockSpec((1, HF), lambda i: (0, 0)),
            pl.BlockSpec((HW, HF), lambda i: (0, 0)),
            pl.BlockSpec((HF, F), lambda i: (0, 0)),
        ],
        out_specs=pl.BlockSpec((bn, F), lambda i: (i, 0)),
        out_shape=jax.ShapeDtypeStruct((n, F), jnp.float32),
    )(acc, den, bexp, e8, m)


def _sc_body(edge_ref, el_ref, er_ref, hflat_ref,
             out_ref, den_ref,
             wsrc, wdst, match, sbuf0, sbuf1, elblk, erblk, wblk,
             srcb, dstb, dlocb, gidxb0, gidxb1, didxb0, didxb1,
             zden, acc_sp, den_sp, sem_g, sem_h, sem_d):
    sbuf = (sbuf0, sbuf1)
    gidxb = (gidxb0, gidxb1)
    didxb = (didxb0, didxb1)
    c = lax.axis_index("c")
    s = lax.axis_index("s")
    iota16 = lax.iota(jnp.int32, 16)
    zero16 = jnp.zeros((16,), jnp.float32)
    lane_is_head = iota16 < H
    base_e = s * ES

    def _zd(i, _):
        def _zc(j, _):
            zden[i, pl.ds(j * 16, 16)] = zero16
            return 0
        return lax.fori_loop(0, HW // 16, _zc, 0)
    lax.fori_loop(0, ROWS_PT, _zd, 0)

    def _zw(i, _):
        def _zc(j, _):
            wblk[i, pl.ds(j * 16, 16)] = zero16
            return 0
        return lax.fori_loop(0, HW // 16, _zc, 0)
    lax.fori_loop(0, BE, _zw, 0)

    def _chunk(k, _):
        lo = (c * CPC + k) * C

        for j in range(H):
            pltpu.sync_copy(zden, acc_sp.at[pl.ds(j * C + s * ROWS_PT, ROWS_PT)])
        pltpu.sync_copy(zden, den_sp.at[pl.ds(s * ROWS_PT, ROWS_PT)])
        plsc.subcore_barrier()

        def _wave(wv_i, _):
            woff = base_e + wv_i * WE
            pltpu.sync_copy(edge_ref.at[pl.ds(woff, WE)], wsrc)
            pltpu.sync_copy(edge_ref.at[pl.ds(E + woff, WE)], wdst)

            @plsc.parallel_loop(0, WG, unroll=4, carry=jnp.int32(0))
            def count(g, cnt):
                d16 = wdst[pl.ds(g * 16, 16)]
                m = (d16 >= lo) & (d16 < lo + C)
                mi = jnp.where(m, 1, 0).astype(jnp.int32)
                cum = plsc.cumsum(mi)
                pos = cnt + cum - 1
                lid = g * 16 + iota16
                plsc.store_scatter(match, [pos], lid, mask=m)
                return cnt + cum[15]

            nb = (count + (BE - 1)) // BE

            def _block(b, _):
                for half in range(BE // 16):
                    off = b * BE + half * 16
                    lid = match[pl.ds(off, 16)]
                    valid = (off + iota16) < count
                    lid = jnp.where(valid, lid, 0)
                    s16 = plsc.load_gather(wsrc, [lid])
                    d16 = plsc.load_gather(wdst, [lid])
                    s16 = jnp.where(valid, s16, 0)
                    d16 = jnp.where(valid, d16, lo)
                    srcb[pl.ds(half * 16, 16)] = s16
                    dstb[pl.ds(half * 16, 16)] = d16
                    dlocb[pl.ds(half * 16, 16)] = d16 - lo
                for g in range(2):
                    for j in range(4):
                        hj = g * 4 + j
                        gi = gidxb[g]
                        di = didxb[g]
                        gi[pl.ds(j * BE, 16)] = srcb[pl.ds(0, 16)] + hj * N
                        gi[pl.ds(j * BE + 16, 16)] = srcb[pl.ds(16, 16)] + hj * N
                        di[pl.ds(j * BE, 16)] = dlocb[pl.ds(0, 16)] + hj * C
                        di[pl.ds(j * BE + 16, 16)] = dlocb[pl.ds(16, 16)] + hj * C
                cp0 = pltpu.async_copy(hflat_ref.at[gidxb[0]], sbuf[0], sem_h)
                cp1 = pltpu.async_copy(hflat_ref.at[gidxb[1]], sbuf[1], sem_h)
                cpe = pltpu.async_copy(el_ref.at[srcb], elblk, sem_g)
                cpr = pltpu.async_copy(er_ref.at[dstb], erblk, sem_g)
                cpe.wait()
                cpr.wait()
                bbase = b * BE
                for e in range(BE):
                    sv = elblk[e, pl.ds(0, 16)] + erblk[e, pl.ds(0, 16)]
                    wv = jnp.exp(jnp.maximum(sv, 0.2 * sv))
                    ok = jnp.logical_and((bbase + e) < count, lane_is_head)
                    wblk[e, pl.ds(0, 16)] = jnp.where(ok, wv, zero16)
                cpd = pltpu.async_copy(wblk, den_sp.at[dlocb], sem_d, add=True)

                for g in range(2):
                    (cp0 if g == 0 else cp1).wait()
                    sb = sbuf[g]
                    for j in range(4):
                        hj = g * 4 + j

                        @plsc.parallel_loop(0, BE, unroll=4)
                        def _sc(e):
                            ws = jnp.full((16,), wblk[e, pl.ds(0, 16)][hj],
                                          jnp.float32)
                            for q in range(F // 16):
                                o = q * 16
                                sb[j * BE + e, pl.ds(o, 16)] = (
                                    sb[j * BE + e, pl.ds(o, 16)] * ws)
                    pltpu.sync_copy(sb, acc_sp.at[didxb[g]], add=True)
                cpd.wait()
                return 0
            lax.fori_loop(0, nb, _block, 0)
            return 0
        lax.fori_loop(0, NW, _wave, 0)

        plsc.subcore_barrier()
        for j in range(H):
            pltpu.sync_copy(acc_sp.at[pl.ds(j * C + s * ROWS_PT, ROWS_PT)],
                            out_ref.at[j, pl.ds(lo + s * ROWS_PT, ROWS_PT)])
        pltpu.sync_copy(den_sp.at[pl.ds(s * ROWS_PT, ROWS_PT)],
                        den_ref.at[pl.ds(lo + s * ROWS_PT, ROWS_PT)])
        return 0
    lax.fori_loop(0, CPC, _chunk, 0)


def _sc_edge(edge_index, el, er, hflat):
    mesh = plsc.VectorSubcoreMesh(core_axis_name="c", subcore_axis_name="s",
                                  num_cores=NCORE, num_subcores=NSUB)
    kern = pl.kernel(
        _sc_body,
        out_type=[
            jax.ShapeDtypeStruct((H, NPAD, HW), jnp.float32),
            jax.ShapeDtypeStruct((NPAD, HW), jnp.float32),
        ],
        mesh=mesh,
        scratch_types=[
            pltpu.VMEM((WE,), jnp.int32),
            pltpu.VMEM((WE,), jnp.int32),
            pltpu.VMEM((WCAP,), jnp.int32),
            pltpu.VMEM((4 * BE, HW), jnp.float32),
            pltpu.VMEM((4 * BE, HW), jnp.float32),
            pltpu.VMEM((BE, HW), jnp.float32),
            pltpu.VMEM((BE, HW), jnp.float32),
            pltpu.VMEM((BE, HW), jnp.float32),
            pltpu.VMEM((BE,), jnp.int32),
            pltpu.VMEM((BE,), jnp.int32),
            pltpu.VMEM((BE,), jnp.int32),
            pltpu.VMEM((4 * BE,), jnp.int32),
            pltpu.VMEM((4 * BE,), jnp.int32),
            pltpu.VMEM((4 * BE,), jnp.int32),
            pltpu.VMEM((4 * BE,), jnp.int32),
            pltpu.VMEM((ROWS_PT, HW), jnp.float32),
            pltpu.VMEM_SHARED((H * C, HW), jnp.float32),
            pltpu.VMEM_SHARED((C, HW), jnp.float32),
            pltpu.SemaphoreType.DMA,
            pltpu.SemaphoreType.DMA,
            pltpu.SemaphoreType.DMA,
        ],
        compiler_params=pltpu.CompilerParams(needs_layout_passes=False),
    )
    return kern(edge_index.reshape(2 * E), el, er, hflat)


def _expand_att(a):
    eye = jnp.eye(HW, dtype=jnp.float32)[:H]
    return (a[:, :, None] * eye[:, None, :]).reshape(HF, HW)


def kernel(x, edge_index, W1, al1, ar1, b1, W2, al2, ar2, b2):
    aal1, aar1 = _expand_att(al1), _expand_att(ar1)
    aal2, aar2 = _expand_att(al2), _expand_att(ar2)
    e8 = jnp.zeros((HW, HF), jnp.float32).at[:H].set(
        jnp.repeat(jnp.eye(H, dtype=jnp.float32), F, axis=1))
    m = jnp.tile(jnp.eye(F, dtype=jnp.float32) / H, (H, 1))
    bexp1 = b1.reshape(1, HF)
    bexp2 = b2.reshape(1, HF)

    h1, el1, er1 = _feats(x, W1, aal1, aar1, bn=1000)
    h1f = jnp.transpose(h1.reshape(N, H, F), (1, 0, 2)).reshape(H * N, F)
    acc1, den1 = _sc_edge(edge_index, el1, er1, h1f)
    acc1t = jnp.transpose(acc1, (1, 0, 2)).reshape(NPAD, HF)
    h2, el2, er2 = _mid(acc1t, den1, bexp1, e8, m, W2, aal2, aar2, bn=1024)
    h2f = jnp.transpose(h2.reshape(NPAD, H, F)[:N], (1, 0, 2)).reshape(H * N, F)
    acc2, den2 = _sc_edge(edge_index, el2, er2, h2f)
    acc2t = jnp.transpose(acc2, (1, 0, 2)).reshape(NPAD, HF)
    out = _final(acc2t, den2, bexp2, e8, m, bn=1024)
    return out[:N]


# speedup vs baseline: 17.6656x; 1.0524x over previous
import functools

import jax
import jax.numpy as jnp
from jax import lax
from jax.experimental import pallas as pl
from jax.experimental.pallas import tpu as pltpu
from jax.experimental.pallas import tpu_sc as plsc

N = 10000
E = 320000
H = 8
F = 128
HF = H * F
HW = 128
NPAD = 10240
NCORE = 2
NSUB = 16
C = 512
CPC = (NPAD // C) // NCORE
ES = E // NSUB
ROWS_PT = C // NSUB
BE = 32
WG = 250
WE = WG * 16
NW = ES // WE
WCAP = 4096


def _feats_body(x_ref, w_ref, aal_ref, aar_ref, h_ref, el_ref, er_ref):
    hb = jnp.dot(x_ref[...], w_ref[...], preferred_element_type=jnp.float32)
    h_ref[...] = hb
    el_ref[...] = jnp.dot(hb, aal_ref[...], preferred_element_type=jnp.float32)
    er_ref[...] = jnp.dot(hb, aar_ref[...], preferred_element_type=jnp.float32)


def _feats(x, w, aal, aar, bn):
    n = x.shape[0]
    return pl.pallas_call(
        _feats_body,
        grid=(n // bn,),
        in_specs=[
            pl.BlockSpec((bn, F), lambda i: (i, 0)),
            pl.BlockSpec((F, HF), lambda i: (0, 0)),
            pl.BlockSpec((HF, HW), lambda i: (0, 0)),
            pl.BlockSpec((HF, HW), lambda i: (0, 0)),
        ],
        out_specs=[
            pl.BlockSpec((bn, HF), lambda i: (i, 0)),
            pl.BlockSpec((bn, HW), lambda i: (i, 0)),
            pl.BlockSpec((bn, HW), lambda i: (i, 0)),
        ],
        out_shape=[
            jax.ShapeDtypeStruct((n, HF), jnp.float32),
            jax.ShapeDtypeStruct((n, HW), jnp.float32),
            jax.ShapeDtypeStruct((n, HW), jnp.float32),
        ],
    )(x, w, aal, aar)


def _mid_body(acc_ref, den_ref, bexp_ref, e8_ref, m_ref,
              w_ref, aal_ref, aar_ref, h_ref, el_ref, er_ref):
    dinv = 1.0 / (den_ref[...] + 1e-9)
    dexp = jnp.dot(dinv, e8_ref[...], preferred_element_type=jnp.float32)
    t = acc_ref[...] * dexp + bexp_ref[...]
    x1 = jnp.dot(t, m_ref[...], preferred_element_type=jnp.float32)
    x1 = jnp.where(x1 > 0.0, x1, jnp.exp(jnp.minimum(x1, 0.0)) - 1.0)
    hb = jnp.dot(x1, w_ref[...], preferred_element_type=jnp.float32)
    h_ref[...] = hb
    el_ref[...] = jnp.dot(hb, aal_ref[...], preferred_element_type=jnp.float32)
    er_ref[...] = jnp.dot(hb, aar_ref[...], preferred_element_type=jnp.float32)


def _mid(acc, den, bexp, e8, m, w, aal, aar, bn):
    n = acc.shape[0]
    return pl.pallas_call(
        _mid_body,
        grid=(n // bn,),
        in_specs=[
            pl.BlockSpec((bn, HF), lambda i: (i, 0)),
            pl.BlockSpec((bn, HW), lambda i: (i, 0)),
            pl.BlockSpec((1, HF), lambda i: (0, 0)),
            pl.BlockSpec((HW, HF), lambda i: (0, 0)),
            pl.BlockSpec((HF, F), lambda i: (0, 0)),
            pl.BlockSpec((F, HF), lambda i: (0, 0)),
            pl.BlockSpec((HF, HW), lambda i: (0, 0)),
            pl.BlockSpec((HF, HW), lambda i: (0, 0)),
        ],
        out_specs=[
            pl.BlockSpec((bn, HF), lambda i: (i, 0)),
            pl.BlockSpec((bn, HW), lambda i: (i, 0)),
            pl.BlockSpec((bn, HW), lambda i: (i, 0)),
        ],
        out_shape=[
            jax.ShapeDtypeStruct((n, HF), jnp.float32),
            jax.ShapeDtypeStruct((n, HW), jnp.float32),
            jax.ShapeDtypeStruct((n, HW), jnp.float32),
        ],
    )(acc, den, bexp, e8, m, w, aal, aar)


def _final_body(acc_ref, den_ref, bexp_ref, e8_ref, m_ref, o_ref):
    dinv = 1.0 / (den_ref[...] + 1e-9)
    dexp = jnp.dot(dinv, e8_ref[...], preferred_element_type=jnp.float32)
    t = acc_ref[...] * dexp + bexp_ref[...]
    o_ref[...] = jnp.dot(t, m_ref[...], preferred_element_type=jnp.float32)


def _final(acc, den, bexp, e8, m, bn):
    n = acc.shape[0]
    return pl.pallas_call(
        _final_body,
        grid=(n // bn,),
        in_specs=[
            pl.BlockSpec((bn, HF), lambda i: (i, 0)),
            pl.BlockSpec((bn, HW), lambda i: (i, 0)),
            pl.BlockSpec((1, HF), lambda i: (0, 0)),
            pl.BlockSpec((HW, HF), lambda i: (0, 0)),
            pl.BlockSpec((HF, F), lambda i: (0, 0)),
        ],
        out_specs=pl.BlockSpec((bn, F), lambda i: (i, 0)),
        out_shape=jax.ShapeDtypeStruct((n, F), jnp.float32),
    )(acc, den, bexp, e8, m)


def _sc_body(edge_ref, el_ref, er_ref, hflat_ref,
             out_ref, den_ref,
             wsrc, wdst, match, sbuf0, sbuf1, elblk, erblk, wblk,
             srcb, dstb, dlocb, gidxb0, gidxb1, didxb0, didxb1,
             zden, acc_sp, den_sp, sem_g, sem_h, sem_d, sem_s):
    sbuf = (sbuf0, sbuf1)
    gidxb = (gidxb0, gidxb1)
    didxb = (didxb0, didxb1)
    c = lax.axis_index("c")
    s = lax.axis_index("s")
    iota16 = lax.iota(jnp.int32, 16)
    zero16 = jnp.zeros((16,), jnp.float32)
    lane_is_head = iota16 < H
    base_e = s * ES

    def _zd(i, _):
        def _zc(j, _):
            zden[i, pl.ds(j * 16, 16)] = zero16
            return 0
        return lax.fori_loop(0, HW // 16, _zc, 0)
    lax.fori_loop(0, ROWS_PT, _zd, 0)

    def _zw(i, _):
        def _zc(j, _):
            wblk[i, pl.ds(j * 16, 16)] = zero16
            return 0
        return lax.fori_loop(0, HW // 16, _zc, 0)
    lax.fori_loop(0, BE, _zw, 0)

    def _chunk(k, _):
        lo = (c * CPC + k) * C

        for j in range(H):
            pltpu.sync_copy(zden, acc_sp.at[pl.ds(j * C + s * ROWS_PT, ROWS_PT)])
        pltpu.sync_copy(zden, den_sp.at[pl.ds(s * ROWS_PT, ROWS_PT)])
        plsc.subcore_barrier()

        def _wave(wv_i, _):
            woff = base_e + wv_i * WE
            pltpu.sync_copy(edge_ref.at[pl.ds(woff, WE)], wsrc)
            pltpu.sync_copy(edge_ref.at[pl.ds(E + woff, WE)], wdst)

            @plsc.parallel_loop(0, WG, unroll=4, carry=jnp.int32(0))
            def count(g, cnt):
                d16 = wdst[pl.ds(g * 16, 16)]
                m = (d16 >= lo) & (d16 < lo + C)
                mi = jnp.where(m, 1, 0).astype(jnp.int32)
                cum = plsc.cumsum(mi)
                pos = cnt + cum - 1
                lid = g * 16 + iota16
                plsc.store_scatter(match, [pos], lid, mask=m)
                return cnt + cum[15]

            nb = (count + (BE - 1)) // BE

            def _block(b, _):
                for half in range(BE // 16):
                    off = b * BE + half * 16
                    lid = match[pl.ds(off, 16)]
                    valid = (off + iota16) < count
                    lid = jnp.where(valid, lid, 0)
                    s16 = plsc.load_gather(wsrc, [lid])
                    d16 = plsc.load_gather(wdst, [lid])
                    s16 = jnp.where(valid, s16, 0)
                    d16 = jnp.where(valid, d16, lo)
                    srcb[pl.ds(half * 16, 16)] = s16
                    dstb[pl.ds(half * 16, 16)] = d16
                    dlocb[pl.ds(half * 16, 16)] = d16 - lo
                for g in range(2):
                    for j in range(4):
                        hj = g * 4 + j
                        gi = gidxb[g]
                        di = didxb[g]
                        gi[pl.ds(j * BE, 16)] = srcb[pl.ds(0, 16)] + hj * N
                        gi[pl.ds(j * BE + 16, 16)] = srcb[pl.ds(16, 16)] + hj * N
                        di[pl.ds(j * BE, 16)] = dlocb[pl.ds(0, 16)] + hj * C
                        di[pl.ds(j * BE + 16, 16)] = dlocb[pl.ds(16, 16)] + hj * C
                cp0 = pltpu.async_copy(hflat_ref.at[gidxb[0]], sbuf[0], sem_h)
                cp1 = pltpu.async_copy(hflat_ref.at[gidxb[1]], sbuf[1], sem_h)
                cpe = pltpu.async_copy(el_ref.at[srcb], elblk, sem_g)
                cpr = pltpu.async_copy(er_ref.at[dstb], erblk, sem_g)
                cpe.wait()
                cpr.wait()
                bbase = b * BE
                for e in range(BE):
                    sv = elblk[e, pl.ds(0, 16)] + erblk[e, pl.ds(0, 16)]
                    wv = jnp.exp(jnp.maximum(sv, 0.2 * sv))
                    ok = jnp.logical_and((bbase + e) < count, lane_is_head)
                    wblk[e, pl.ds(0, 16)] = jnp.where(ok, wv, zero16)
                cpd = pltpu.async_copy(wblk, den_sp.at[dlocb], sem_d, add=True)

                cps = []
                for g in range(2):
                    (cp0 if g == 0 else cp1).wait()
                    sb = sbuf[g]
                    for j in range(4):
                        hj = g * 4 + j

                        @plsc.parallel_loop(0, BE, unroll=4)
                        def _sc(e):
                            ws = jnp.full((16,), wblk[e, pl.ds(0, 16)][hj],
                                          jnp.float32)
                            for q in range(F // 16):
                                o = q * 16
                                sb[j * BE + e, pl.ds(o, 16)] = (
                                    sb[j * BE + e, pl.ds(o, 16)] * ws)
                    cps.append(
                        pltpu.async_copy(sb, acc_sp.at[didxb[g]], sem_s,
                                         add=True))
                for cp in cps:
                    cp.wait()
                cpd.wait()
                return 0
            lax.fori_loop(0, nb, _block, 0)
            return 0
        lax.fori_loop(0, NW, _wave, 0)

        plsc.subcore_barrier()
        for j in range(H):
            pltpu.sync_copy(acc_sp.at[pl.ds(j * C + s * ROWS_PT, ROWS_PT)],
                            out_ref.at[j, pl.ds(lo + s * ROWS_PT, ROWS_PT)])
        pltpu.sync_copy(den_sp.at[pl.ds(s * ROWS_PT, ROWS_PT)],
                        den_ref.at[pl.ds(lo + s * ROWS_PT, ROWS_PT)])
        return 0
    lax.fori_loop(0, CPC, _chunk, 0)


def _sc_edge(edge_index, el, er, hflat):
    mesh = plsc.VectorSubcoreMesh(core_axis_name="c", subcore_axis_name="s",
                                  num_cores=NCORE, num_subcores=NSUB)
    kern = pl.kernel(
        _sc_body,
        out_type=[
            jax.ShapeDtypeStruct((H, NPAD, HW), jnp.float32),
            jax.ShapeDtypeStruct((NPAD, HW), jnp.float32),
        ],
        mesh=mesh,
        scratch_types=[
            pltpu.VMEM((WE,), jnp.int32),
            pltpu.VMEM((WE,), jnp.int32),
            pltpu.VMEM((WCAP,), jnp.int32),
            pltpu.VMEM((4 * BE, HW), jnp.float32),
            pltpu.VMEM((4 * BE, HW), jnp.float32),
            pltpu.VMEM((BE, HW), jnp.float32),
            pltpu.VMEM((BE, HW), jnp.float32),
            pltpu.VMEM((BE, HW), jnp.float32),
            pltpu.VMEM((BE,), jnp.int32),
            pltpu.VMEM((BE,), jnp.int32),
            pltpu.VMEM((BE,), jnp.int32),
            pltpu.VMEM((4 * BE,), jnp.int32),
            pltpu.VMEM((4 * BE,), jnp.int32),
            pltpu.VMEM((4 * BE,), jnp.int32),
            pltpu.VMEM((4 * BE,), jnp.int32),
            pltpu.VMEM((ROWS_PT, HW), jnp.float32),
            pltpu.VMEM_SHARED((H * C, HW), jnp.float32),
            pltpu.VMEM_SHARED((C, HW), jnp.float32),
            pltpu.SemaphoreType.DMA,
            pltpu.SemaphoreType.DMA,
            pltpu.SemaphoreType.DMA,
            pltpu.SemaphoreType.DMA,
        ],
        compiler_params=pltpu.CompilerParams(needs_layout_passes=False),
    )
    return kern(edge_index.reshape(2 * E), el, er, hflat)


def _expand_att(a):
    eye = jnp.eye(HW, dtype=jnp.float32)[:H]
    return (a[:, :, None] * eye[:, None, :]).reshape(HF, HW)


def kernel(x, edge_index, W1, al1, ar1, b1, W2, al2, ar2, b2):
    aal1, aar1 = _expand_att(al1), _expand_att(ar1)
    aal2, aar2 = _expand_att(al2), _expand_att(ar2)
    e8 = jnp.zeros((HW, HF), jnp.float32).at[:H].set(
        jnp.repeat(jnp.eye(H, dtype=jnp.float32), F, axis=1))
    m = jnp.tile(jnp.eye(F, dtype=jnp.float32) / H, (H, 1))
    bexp1 = b1.reshape(1, HF)
    bexp2 = b2.reshape(1, HF)

    h1, el1, er1 = _feats(x, W1, aal1, aar1, bn=1000)
    h1f = jnp.transpose(h1.reshape(N, H, F), (1, 0, 2)).reshape(H * N, F)
    acc1, den1 = _sc_edge(edge_index, el1, er1, h1f)
    acc1t = jnp.transpose(acc1, (1, 0, 2)).reshape(NPAD, HF)
    h2, el2, er2 = _mid(acc1t, den1, bexp1, e8, m, W2, aal2, aar2, bn=1024)
    h2f = jnp.transpose(h2.reshape(NPAD, H, F)[:N], (1, 0, 2)).reshape(H * N, F)
    acc2, den2 = _sc_edge(edge_index, el2, er2, h2f)
    acc2t = jnp.transpose(acc2, (1, 0, 2)).reshape(NPAD, HF)
    out = _final(acc2t, den2, bexp2, e8, m, bn=1024)
    return out[:N]
